# baseline (speedup 1.0000x reference)
import functools

import jax
import jax.numpy as jnp
from jax.experimental import pallas as pl

_P = 0.5
_X = 0.5


@functools.lru_cache(maxsize=None)
def _mask_int8(bsz: int, rows: int, cols: int):
    key = jax.random.key(42)
    k_perm, k_sel = jax.random.split(key)
    n_sel = int(bsz * _X)
    indices = jax.random.permutation(k_perm, bsz)[:n_sel]
    select = jax.random.uniform(k_sel, (n_sel, rows, cols), dtype=jnp.float32)
    keep_sel = (select > _P)
    full = jnp.ones((bsz, rows, cols), dtype=jnp.bool_).at[indices].set(keep_sel)
    return jax.device_put(full.astype(jnp.int8))


def _mul_kernel(x_ref, m_ref, o_ref):
    o_ref[...] = x_ref[...] * m_ref[...].astype(x_ref.dtype)


def kernel(tensor):
    bsz, rows, cols = tensor.shape
    mask = _mask_int8(bsz, rows, cols)

    block_b = 64
    while bsz % block_b:
        block_b //= 2
    grid = (bsz // block_b,)

    return pl.pallas_call(
        _mul_kernel,
        grid=grid,
        in_specs=[
            pl.BlockSpec((block_b, rows, cols), lambda i: (i, 0, 0)),
            pl.BlockSpec((block_b, rows, cols), lambda i: (i, 0, 0)),
        ],
        out_specs=pl.BlockSpec((block_b, rows, cols), lambda i: (i, 0, 0)),
        out_shape=jax.ShapeDtypeStruct((bsz, rows, cols), tensor.dtype),
    )(tensor, mask)


# speedup vs baseline: 2.3273x; 2.3273x over previous
import functools

import jax
import jax.numpy as jnp
from jax.experimental import pallas as pl

_P = 0.5
_X = 0.5


@functools.lru_cache(maxsize=None)
def _mask_int8(bsz: int, rows: int, cols: int):
    with jax.ensure_compile_time_eval():
        key = jax.random.key(42)
        k_perm, k_sel = jax.random.split(key)
        n_sel = int(bsz * _X)
        indices = jax.random.permutation(k_perm, bsz)[:n_sel]
        select = jax.random.uniform(k_sel, (n_sel, rows, cols), dtype=jnp.float32)
        keep_sel = (select > _P)
        full = jnp.ones((bsz, rows, cols), dtype=jnp.bool_).at[indices].set(keep_sel)
        return jax.device_put(full.astype(jnp.int8))


def _mul_kernel(x_ref, m_ref, o_ref):
    o_ref[...] = x_ref[...] * m_ref[...].astype(x_ref.dtype)


def kernel(tensor):
    bsz, rows, cols = tensor.shape
    mask = _mask_int8(bsz, rows, cols)

    block_b = 64
    while bsz % block_b:
        block_b //= 2
    grid = (bsz // block_b,)

    return pl.pallas_call(
        _mul_kernel,
        grid=grid,
        in_specs=[
            pl.BlockSpec((block_b, rows, cols), lambda i: (i, 0, 0)),
            pl.BlockSpec((block_b, rows, cols), lambda i: (i, 0, 0)),
        ],
        out_specs=pl.BlockSpec((block_b, rows, cols), lambda i: (i, 0, 0)),
        out_shape=jax.ShapeDtypeStruct((bsz, rows, cols), tensor.dtype),
    )(tensor, mask)


# speedup vs baseline: 12.6130x; 5.4195x over previous
import functools

import jax
import jax.numpy as jnp
from jax.experimental import pallas as pl

_P = 0.5
_X = 0.5


@functools.lru_cache(maxsize=None)
def _mask_t_int8(bsz: int, rows: int, cols: int):
    with jax.ensure_compile_time_eval():
        key = jax.random.key(42)
        k_perm, k_sel = jax.random.split(key)
        n_sel = int(bsz * _X)
        indices = jax.random.permutation(k_perm, bsz)[:n_sel]
        select = jax.random.uniform(k_sel, (n_sel, rows, cols), dtype=jnp.float32)
        keep_sel = (select > _P)
        full = jnp.ones((bsz, rows, cols), dtype=jnp.bool_).at[indices].set(keep_sel)
        full_t = full.transpose(1, 2, 0).reshape(rows * cols, bsz)
        return jax.device_put(full_t.astype(jnp.int8))


def _mul_kernel(x_ref, m_ref, o_ref):
    o_ref[...] = x_ref[...] * m_ref[...].astype(x_ref.dtype)


def kernel(tensor):
    bsz, rows, cols = tensor.shape
    seq = rows * cols
    mask_t = _mask_t_int8(bsz, rows, cols)
    x_t = tensor.transpose(1, 2, 0).reshape(seq, bsz)

    block_s = 512
    while seq % block_s:
        block_s //= 2
    grid = (seq // block_s,)

    out_t = pl.pallas_call(
        _mul_kernel,
        grid=grid,
        in_specs=[
            pl.BlockSpec((block_s, bsz), lambda i: (i, 0)),
            pl.BlockSpec((block_s, bsz), lambda i: (i, 0)),
        ],
        out_specs=pl.BlockSpec((block_s, bsz), lambda i: (i, 0)),
        out_shape=jax.ShapeDtypeStruct((seq, bsz), tensor.dtype),
    )(x_t, mask_t)
    return out_t.reshape(rows, cols, bsz).transpose(2, 0, 1)
